# baseline (speedup 1.0000x reference)
import functools

import jax
import jax.numpy as jnp
from jax import lax
from jax.experimental import pallas as pl
from jax.experimental.pallas import tpu as pltpu
from jax.experimental.pallas import tpu_sc as plsc

BATCH = 4096
HIST = 200
EMB = 64
NCLS = 100

NC = 2
NS = 16
NW = NC * NS
B_PER_W = BATCH // NW
NBUF = 4
SPLIT0 = 104
SPLIT1 = HIST - SPLIT0
NFULL = HIST // 16
NREM = HIST - NFULL * 16


def _pool_kernel(x_hbm, emb_hbm, out_hbm,
                 idx_v, g0, g1, g2, g3, out_v,
                 s0, s1, s2, s3):
    gbufs = (g0, g1, g2, g3)
    sems = (s0, s1, s2, s3)
    wid = lax.axis_index("s") * NC + lax.axis_index("c")
    base = wid * B_PER_W

    pltpu.sync_copy(x_hbm.at[pl.ds(base * HIST, B_PER_W * HIST)], idx_v)

    def fire(b, slot):
        off = b * HIST
        pltpu.async_copy(emb_hbm.at[idx_v.at[pl.ds(off, SPLIT0)]],
                         gbufs[slot].at[pl.ds(0, SPLIT0)], sems[slot])
        pltpu.async_copy(emb_hbm.at[idx_v.at[pl.ds(off + SPLIT0, SPLIT1)]],
                         gbufs[slot].at[pl.ds(SPLIT0, SPLIT1)], sems[slot])

    def drain(slot):
        pltpu.make_async_copy(emb_hbm.at[pl.ds(0, HIST)], gbufs[slot],
                              sems[slot]).wait()

    def consume(b, slot):
        gb = gbufs[slot]

        def rows(i, acc):
            a0, a1, a2, a3 = acc
            r = i * 4
            for k in range(4):
                a0 = a0 + gb[r + k, pl.ds(0, 16)]
                a1 = a1 + gb[r + k, pl.ds(16, 16)]
                a2 = a2 + gb[r + k, pl.ds(32, 16)]
                a3 = a3 + gb[r + k, pl.ds(48, 16)]
            return (a0, a1, a2, a3)
        zero = jnp.zeros((16,), jnp.float32)
        acc = lax.fori_loop(0, HIST // 4, rows, (zero, zero, zero, zero))

        for c in range(EMB // 16):
            out_v[pl.ds(b * EMB + c * 16, 16)] = acc[c]

    for s in range(NBUF - 1):
        fire(s, s)

    def group(g, carry):
        for s in range(NBUF):
            b = g * NBUF + s
            nb = b + NBUF - 1
            nslot = (s + NBUF - 1) % NBUF

            @pl.when(nb < B_PER_W)
            def _():
                fire(nb, nslot)

            drain(s)
            consume(b, s)
        return carry
    lax.fori_loop(0, B_PER_W // NBUF, group, 0)

    pltpu.sync_copy(out_v, out_hbm.at[pl.ds(base * EMB, B_PER_W * EMB)])


_pool = functools.partial(
    pl.kernel,
    out_type=jax.ShapeDtypeStruct((BATCH * EMB,), jnp.float32),
    mesh=plsc.VectorSubcoreMesh(core_axis_name="c", subcore_axis_name="s"),
    compiler_params=pltpu.CompilerParams(use_tc_tiling_on_sc=False),
    scratch_types=[
        pltpu.VMEM((B_PER_W * HIST,), jnp.int32),
        pltpu.VMEM((HIST, EMB), jnp.float32),
        pltpu.VMEM((HIST, EMB), jnp.float32),
        pltpu.VMEM((HIST, EMB), jnp.float32),
        pltpu.VMEM((HIST, EMB), jnp.float32),
        pltpu.VMEM((B_PER_W * EMB,), jnp.float32),
        pltpu.SemaphoreType.DMA,
        pltpu.SemaphoreType.DMA,
        pltpu.SemaphoreType.DMA,
        pltpu.SemaphoreType.DMA,
    ],
)(_pool_kernel)


def _mm_body(m_ref, x_ref, e0_ref, w_ref, b_ref, o_ref):
    mm = lax.dot_general(
        m_ref[...], w_ref[...], (((1,), (1,)), ((), ())),
        preferred_element_type=jnp.float32)
    e0w = lax.dot_general(
        e0_ref[...], w_ref[...], (((1,), (1,)), ((), ())),
        preferred_element_type=jnp.float32)
    cnt = jnp.sum((x_ref[...] == 0).astype(jnp.float32), axis=1,
                  keepdims=True)
    o_ref[...] = (mm - cnt * e0w) * (1.0 / HIST) + b_ref[...]


def _classify(m, x, e0, W, b):
    return pl.pallas_call(
        _mm_body,
        out_shape=jax.ShapeDtypeStruct((BATCH, NCLS), jnp.float32),
    )(m, x, e0, W, b.reshape(1, NCLS))


def kernel(x, emb, W, b):
    pooled = _pool(x.reshape(-1), emb)
    m = pooled.reshape(BATCH, EMB)
    return _classify(m, x, emb[0:1, :], W, b)
